# Initial kernel scaffold; baseline (speedup 1.0000x reference)
import jax
import jax.numpy as jnp
from jax.experimental import pallas as pl


def kernel(x, edge_index, W1, b1, W_out, b_out):
    raise NotImplementedError("write your pallas kernel here")


# speedup vs baseline: 41.6342x; 41.6342x over previous
import functools

import jax
import jax.numpy as jnp
from jax import lax
from jax.experimental import pallas as pl
from jax.experimental.pallas import tpu as pltpu
from jax.experimental.pallas import tpu_sc as plsc

_N = 10000
_E = 320000
_D = 128
_H = 128

_NW = 32
_LANE = 128
_CHUNKS = -(-_E // (_NW * _LANE))
_CAP = _NW * _CHUNKS * _LANE
_NBINS = 10240
_PER_TILE = _NBINS // 16

@functools.cache
def _make_sc_histogram():
    mesh = plsc.VectorSubcoreMesh(core_axis_name="c", subcore_axis_name="s")
    return functools.partial(
        pl.kernel,
        out_type=jax.ShapeDtypeStruct((2, _NBINS), jnp.float32),
        mesh=mesh,
        scratch_types=[
            pltpu.VMEM((_CHUNKS, _LANE), jnp.int32),
            pltpu.VMEM((_LANE,), jnp.float32),
            pltpu.VMEM((_PER_TILE,), jnp.float32),
            pltpu.VMEM_SHARED((_NBINS,), jnp.float32),
        ],
    )(_sc_histogram_body)


def _sc_histogram_body(src_hbm, out_hbm, idx_v, ones_v, zeros_v, counts_sh):
    cid = lax.axis_index("c")
    sid = lax.axis_index("s")
    wid = sid * 2 + cid

    one16 = jnp.ones((16,), jnp.float32)
    zero16 = jnp.zeros((16,), jnp.float32)
    for i in range(_LANE // 16):
        ones_v[pl.ds(i * 16, 16)] = one16
    for i in range(_PER_TILE // 16):
        zeros_v[pl.ds(i * 16, 16)] = zero16

    pltpu.sync_copy(zeros_v, counts_sh.at[pl.ds(sid * _PER_TILE, _PER_TILE)])
    pltpu.sync_copy(src_hbm.at[wid], idx_v)
    plsc.subcore_barrier()

    def body(j, carry):
        pltpu.sync_copy(ones_v, counts_sh.at[idx_v.at[j]], add=True)
        return carry

    lax.fori_loop(0, _CHUNKS, body, 0)
    plsc.subcore_barrier()

    @pl.when(sid == 0)
    def _():
        pltpu.sync_copy(counts_sh, out_hbm.at[cid])


def _tc_head(counts_ref, x_ref, w1_ref, b1_ref, wout_ref, bout_ref, o_ref):
    dn = (((1,), (0,)), ((), ()))
    hi = jax.lax.Precision.HIGHEST
    c = counts_ref[...]
    c1 = c[0:1, :] + c[1:2, :]
    s = lax.dot_general(c1, x_ref[...], dn, precision=hi,
                        preferred_element_type=jnp.float32)
    mol = lax.dot_general(s, w1_ref[...], dn, precision=hi,
                          preferred_element_type=jnp.float32)
    mol = mol + jnp.float32(_N) * b1_ref[...]
    out = lax.dot_general(mol, wout_ref[...], dn, precision=hi,
                          preferred_element_type=jnp.float32)
    o_ref[...] = out + bout_ref[...]


def kernel(x, edge_index, W1, b1, W_out, b_out):
    src = edge_index[0]
    pad = (jnp.arange(_CAP - _E, dtype=jnp.int32) % (_NBINS - _N)) + _N
    src_p = jnp.concatenate([src, pad]).reshape(_NW, _CHUNKS, _LANE)
    x_p = jnp.pad(x, ((0, _NBINS - _N), (0, 0)))

    counts = _make_sc_histogram()(src_p)

    out = pl.pallas_call(
        _tc_head,
        out_shape=jax.ShapeDtypeStruct((1, 1), jnp.float32),
    )(counts, x_p, W1, b1.reshape(1, _H), W_out, b_out.reshape(1, 1))
    return out


# speedup vs baseline: 42.7543x; 1.0269x over previous
import functools

import jax
import jax.numpy as jnp
from jax import lax
from jax.experimental import pallas as pl
from jax.experimental.pallas import tpu as pltpu
from jax.experimental.pallas import tpu_sc as plsc

_N = 10000
_E = 320000
_D = 128
_H = 128

_NW = 32
_LANE = 128
_CHUNKS = -(-_E // (_NW * _LANE))
_CAP = _NW * _CHUNKS * _LANE
_NBINS = 10240
_PER_TILE = _NBINS // 16

@functools.cache
def _make_sc_histogram():
    mesh = plsc.VectorSubcoreMesh(core_axis_name="c", subcore_axis_name="s")
    return functools.partial(
        pl.kernel,
        out_type=jax.ShapeDtypeStruct((2, _NBINS), jnp.float32),
        mesh=mesh,
        scratch_types=[
            pltpu.VMEM((_CHUNKS * _LANE,), jnp.int32),
            pltpu.VMEM((_CHUNKS * _LANE,), jnp.float32),
            pltpu.VMEM((_PER_TILE,), jnp.float32),
            pltpu.VMEM_SHARED((_NBINS,), jnp.float32),
        ],
    )(_sc_histogram_body)


def _sc_histogram_body(src_hbm, ones_hbm, out_hbm, idx_v, ones_v, zeros_v,
                       counts_sh):
    cid = lax.axis_index("c")
    sid = lax.axis_index("s")
    wid = sid * 2 + cid

    zero16 = jnp.zeros((16,), jnp.float32)
    for i in range(_PER_TILE // 16):
        zeros_v[pl.ds(i * 16, 16)] = zero16

    pltpu.sync_copy(zeros_v, counts_sh.at[pl.ds(sid * _PER_TILE, _PER_TILE)])
    pltpu.sync_copy(src_hbm.at[wid], idx_v)
    pltpu.sync_copy(ones_hbm, ones_v)
    plsc.subcore_barrier()

    pltpu.sync_copy(ones_v, counts_sh.at[idx_v], add=True)
    plsc.subcore_barrier()

    @pl.when(sid == 0)
    def _():
        pltpu.sync_copy(counts_sh, out_hbm.at[cid])


def _tc_head(counts_ref, x_ref, w1_ref, b1_ref, wout_ref, bout_ref, o_ref):
    dn = (((1,), (0,)), ((), ()))
    hi = jax.lax.Precision.HIGHEST
    c = counts_ref[...]
    c1 = c[0:1, :] + c[1:2, :]
    xv = x_ref[...].astype(jnp.float32)
    w1 = w1_ref[...].astype(jnp.float32)
    s = lax.dot_general(c1, xv, dn, precision=hi,
                        preferred_element_type=jnp.float32)
    mol = lax.dot_general(s, w1, dn, precision=hi,
                          preferred_element_type=jnp.float32)
    mol = mol + jnp.float32(_N) * b1_ref[...]
    out = lax.dot_general(mol, wout_ref[...], dn, precision=hi,
                          preferred_element_type=jnp.float32)
    o_ref[...] = out + bout_ref[...]


def kernel(x, edge_index, W1, b1, W_out, b_out):
    src = edge_index[0]
    pad = (jnp.arange(_CAP - _E, dtype=jnp.int32) % (_NBINS - _N)) + _N
    src_p = jnp.concatenate([src, pad]).reshape(_NW, _CHUNKS * _LANE)
    x_p = jnp.pad(x.astype(jnp.bfloat16), ((0, _NBINS - _N), (0, 0)))

    ones = jnp.ones((_CHUNKS * _LANE,), jnp.float32)
    counts = _make_sc_histogram()(src_p, ones)

    out = pl.pallas_call(
        _tc_head,
        out_shape=jax.ShapeDtypeStruct((1, 1), jnp.float32),
    )(counts, x_p, W1.astype(jnp.bfloat16), b1.reshape(1, _H), W_out,
      b_out.reshape(1, 1))
    return out


# speedup vs baseline: 56.8862x; 1.3305x over previous
import functools

import jax
import jax.numpy as jnp
from jax import lax
from jax.experimental import pallas as pl
from jax.experimental.pallas import tpu as pltpu
from jax.experimental.pallas import tpu_sc as plsc

_N = 10000
_E = 320000
_D = 128
_H = 128

_NW = 32
_EPW = _E // _NW
_NBINS = 10240
_PER_TILE = _NBINS // 16


@functools.cache
def _make_sc_histogram():
    mesh = plsc.VectorSubcoreMesh(core_axis_name="c", subcore_axis_name="s")
    return functools.partial(
        pl.kernel,
        out_type=jax.ShapeDtypeStruct((2, _NBINS), jnp.float32),
        mesh=mesh,
        scratch_types=[
            pltpu.VMEM((_EPW,), jnp.int32),
            pltpu.VMEM((_EPW,), jnp.float32),
            pltpu.VMEM((_PER_TILE,), jnp.float32),
            pltpu.VMEM_SHARED((_NBINS,), jnp.float32),
        ],
    )(_sc_histogram_body)


def _sc_histogram_body(edge_hbm, ones_hbm, out_hbm, idx_v, ones_v, zeros_v,
                       counts_sh):
    cid = lax.axis_index("c")
    sid = lax.axis_index("s")
    wid = sid * 2 + cid

    zero16 = jnp.zeros((16,), jnp.float32)
    for i in range(_PER_TILE // 16):
        zeros_v[pl.ds(i * 16, 16)] = zero16

    pltpu.sync_copy(zeros_v, counts_sh.at[pl.ds(sid * _PER_TILE, _PER_TILE)])
    pltpu.sync_copy(edge_hbm.at[pl.ds(wid * _EPW, _EPW)], idx_v)
    pltpu.sync_copy(ones_hbm, ones_v)
    plsc.subcore_barrier()

    pltpu.sync_copy(ones_v, counts_sh.at[idx_v], add=True)
    plsc.subcore_barrier()

    @pl.when(sid == 0)
    def _():
        pltpu.sync_copy(counts_sh, out_hbm.at[cid])


def _tc_head(counts_ref, x_ref, w1_ref, b1_ref, wout_ref, bout_ref, o_ref):
    dn = (((1,), (0,)), ((), ()))
    hi = jax.lax.Precision.HIGHEST
    c = counts_ref[...]
    c1 = (c[0:1, :] + c[1:2, :])[:, :_N]
    xv = x_ref[...].astype(jnp.bfloat16).astype(jnp.float32)
    w1 = w1_ref[...].astype(jnp.bfloat16).astype(jnp.float32)
    s = lax.dot_general(c1, xv, dn, precision=hi,
                        preferred_element_type=jnp.float32)
    mol = lax.dot_general(s, w1, dn, precision=hi,
                          preferred_element_type=jnp.float32)
    mol = mol + jnp.float32(_N) * b1_ref[...]
    out = lax.dot_general(mol, wout_ref[...], dn, precision=hi,
                          preferred_element_type=jnp.float32)
    o_ref[...] = out + bout_ref[...]


def kernel(x, edge_index, W1, b1, W_out, b_out):
    ones = jnp.ones((_EPW,), jnp.float32)
    counts = _make_sc_histogram()(edge_index.reshape(2 * _E), ones)

    out = pl.pallas_call(
        _tc_head,
        out_shape=jax.ShapeDtypeStruct((1, 1), jnp.float32),
    )(counts, x, W1, b1.reshape(1, _H), W_out, b_out.reshape(1, 1))
    return out


# speedup vs baseline: 62.4566x; 1.0979x over previous
import functools

import jax
import jax.numpy as jnp
from jax import lax
from jax.experimental import pallas as pl
from jax.experimental.pallas import tpu as pltpu
from jax.experimental.pallas import tpu_sc as plsc

_N = 10000
_E = 320000
_D = 128
_H = 128

_NW = 32
_EPW = _E // _NW
_NBINS = 10240
_PER_TILE = _NBINS // 16


@functools.cache
def _make_sc_histogram():
    mesh = plsc.VectorSubcoreMesh(core_axis_name="c", subcore_axis_name="s")
    return functools.partial(
        pl.kernel,
        out_type=jax.ShapeDtypeStruct((2, _NBINS), jnp.float32),
        mesh=mesh,
        scratch_types=[
            pltpu.VMEM((_EPW,), jnp.int32),
            pltpu.VMEM((_EPW,), jnp.float32),
            pltpu.VMEM((_PER_TILE,), jnp.float32),
            pltpu.VMEM_SHARED((_NBINS,), jnp.float32),
        ],
    )(_sc_histogram_body)


def _sc_histogram_body(edge_hbm, ones_hbm, out_hbm, idx_v, ones_v, zeros_v,
                       counts_sh):
    cid = lax.axis_index("c")
    sid = lax.axis_index("s")
    wid = sid * 2 + cid

    zero16 = jnp.zeros((16,), jnp.float32)
    for i in range(_PER_TILE // 16):
        zeros_v[pl.ds(i * 16, 16)] = zero16

    pltpu.sync_copy(zeros_v, counts_sh.at[pl.ds(sid * _PER_TILE, _PER_TILE)])
    pltpu.sync_copy(edge_hbm.at[pl.ds(wid * _EPW, _EPW)], idx_v)
    pltpu.sync_copy(ones_hbm, ones_v)
    plsc.subcore_barrier()

    pltpu.sync_copy(ones_v, counts_sh.at[idx_v], add=True)
    plsc.subcore_barrier()

    @pl.when(sid == 0)
    def _():
        pltpu.sync_copy(counts_sh, out_hbm.at[cid])


def _tc_node_head(x_ref, w1_ref, woutt_ref, y_ref):
    hi = jax.lax.Precision.HIGHEST
    xv = x_ref[...].astype(jnp.bfloat16).astype(jnp.float32)
    w1 = w1_ref[...].astype(jnp.bfloat16).astype(jnp.float32)
    w_row = lax.dot_general(woutt_ref[...], w1, (((1,), (1,)), ((), ())),
                            precision=hi,
                            preferred_element_type=jnp.float32)
    y_ref[...] = lax.dot_general(w_row, xv, (((1,), (1,)), ((), ())),
                                 precision=hi,
                                 preferred_element_type=jnp.float32)


def _tc_final(counts_ref, y_ref, b1_ref, wout_ref, bout_ref, o_ref):
    hi = jax.lax.Precision.HIGHEST
    c = counts_ref[...]
    c1 = (c[0:1, :] + c[1:2, :])[:, :_N]
    out = lax.dot_general(c1, y_ref[...], (((1,), (1,)), ((), ())),
                          precision=hi, preferred_element_type=jnp.float32)
    bias = lax.dot_general(b1_ref[...], wout_ref[...],
                           (((1,), (1,)), ((), ())), precision=hi,
                           preferred_element_type=jnp.float32)
    o_ref[...] = out + jnp.float32(_N) * bias + bout_ref[...]


def kernel(x, edge_index, W1, b1, W_out, b_out):
    ones = jnp.ones((_EPW,), jnp.float32)
    counts = _make_sc_histogram()(edge_index.reshape(2 * _E), ones)

    y = pl.pallas_call(
        _tc_node_head,
        out_shape=jax.ShapeDtypeStruct((1, _N), jnp.float32),
    )(x, W1, W_out.reshape(1, _H))

    out = pl.pallas_call(
        _tc_final,
        out_shape=jax.ShapeDtypeStruct((1, 1), jnp.float32),
    )(counts, y, b1.reshape(1, _H), W_out.reshape(1, _H),
      b_out.reshape(1, 1))
    return out


# speedup vs baseline: 64.0501x; 1.0255x over previous
import functools

import jax
import jax.numpy as jnp
from jax import lax
from jax.experimental import pallas as pl
from jax.experimental.pallas import tpu as pltpu
from jax.experimental.pallas import tpu_sc as plsc

_N = 10000
_E = 320000
_D = 128
_H = 128

_NW = 32
_EPW = _E // _NW
_NBINS = 10240
_PER_TILE = _NBINS // 16


_EPW_PAD = 10240


@functools.cache
def _make_sc_histogram():
    mesh = plsc.VectorSubcoreMesh(core_axis_name="c", subcore_axis_name="s")
    return functools.partial(
        pl.kernel,
        out_type=(jax.ShapeDtypeStruct((_NBINS,), jnp.float32),
                  jax.ShapeDtypeStruct((_NBINS,), jnp.float32)),
        mesh=mesh,
        scratch_types=[
            pltpu.VMEM((_EPW_PAD,), jnp.int32),
            pltpu.VMEM((_EPW_PAD,), jnp.float32),
            pltpu.VMEM((_PER_TILE,), jnp.float32),
            pltpu.VMEM_SHARED((_NBINS,), jnp.float32),
        ],
    )(_sc_histogram_body)


def _sc_histogram_body(edge_hbm, out0_hbm, out1_hbm, idx_v, ones_v, zeros_v,
                       counts_sh):
    cid = lax.axis_index("c")
    sid = lax.axis_index("s")
    wid = sid * 2 + cid

    one16 = jnp.ones((16,), jnp.float32)
    zero16 = jnp.zeros((16,), jnp.float32)
    for i in range(_EPW_PAD // 16):
        ones_v[pl.ds(i * 16, 16)] = one16
    for i in range(_PER_TILE // 16):
        zeros_v[pl.ds(i * 16, 16)] = zero16
    lanes = lax.iota(jnp.int32, 16)
    for i in range((_EPW_PAD - _EPW) // 16):
        idx_v[pl.ds(_EPW + i * 16, 16)] = _N + 256 + i * 16 + lanes

    pltpu.sync_copy(zeros_v, counts_sh.at[pl.ds(sid * _PER_TILE, _PER_TILE)])
    pltpu.sync_copy(edge_hbm.at[pl.ds(wid * _EPW, _EPW)],
                    idx_v.at[pl.ds(0, _EPW)])
    plsc.subcore_barrier()

    pltpu.sync_copy(ones_v, counts_sh.at[idx_v], add=True)
    plsc.subcore_barrier()

    @pl.when(cid == 0)
    def _():
        pltpu.sync_copy(counts_sh.at[pl.ds(sid * _PER_TILE, _PER_TILE)],
                        out0_hbm.at[pl.ds(sid * _PER_TILE, _PER_TILE)])

    @pl.when(cid == 1)
    def _():
        pltpu.sync_copy(counts_sh.at[pl.ds(sid * _PER_TILE, _PER_TILE)],
                        out1_hbm.at[pl.ds(sid * _PER_TILE, _PER_TILE)])


def _tc_node_head(x_ref, w1_ref, woutt_ref, y_ref):
    hi = jax.lax.Precision.HIGHEST
    xv = x_ref[...].astype(jnp.bfloat16).astype(jnp.float32)
    w1 = w1_ref[...].astype(jnp.bfloat16).astype(jnp.float32)
    w_row = lax.dot_general(woutt_ref[...], w1, (((1,), (1,)), ((), ())),
                            precision=hi,
                            preferred_element_type=jnp.float32)
    y_ref[...] = lax.dot_general(w_row, xv, (((1,), (1,)), ((), ())),
                                 precision=hi,
                                 preferred_element_type=jnp.float32)


def _tc_final(c_ref, y_ref, b1_ref, wout_ref, bout_ref, o_ref):
    hi = jax.lax.Precision.HIGHEST
    c1 = c_ref[...][:, :_N]
    out = lax.dot_general(c1, y_ref[...], (((1,), (1,)), ((), ())),
                          precision=hi, preferred_element_type=jnp.float32)
    bias = lax.dot_general(b1_ref[...], wout_ref[...],
                           (((1,), (1,)), ((), ())), precision=hi,
                           preferred_element_type=jnp.float32)
    o_ref[...] = out + jnp.float32(_N) * bias + bout_ref[...]


def kernel(x, edge_index, W1, b1, W_out, b_out):
    counts0, counts1 = _make_sc_histogram()(edge_index.reshape(2 * _E))

    y = pl.pallas_call(
        _tc_node_head,
        out_shape=jax.ShapeDtypeStruct((1, _N), jnp.float32),
    )(x, W1, W_out.reshape(1, _H))

    cf = (counts0.astype(jnp.float32)
          + counts1.astype(jnp.float32)).reshape(1, _NBINS)

    out = pl.pallas_call(
        _tc_final,
        out_shape=jax.ShapeDtypeStruct((1, 1), jnp.float32),
    )(cf, y, b1.reshape(1, _H), W_out.reshape(1, _H), b_out.reshape(1, 1))
    return out


# speedup vs baseline: 67.0265x; 1.0465x over previous
import functools

import jax
import jax.numpy as jnp
from jax import lax
from jax.experimental import pallas as pl
from jax.experimental.pallas import tpu as pltpu
from jax.experimental.pallas import tpu_sc as plsc

_N = 10000
_E = 320000
_D = 128
_H = 128

_NW = 32
_EPW = _E // _NW
_NBINS = 10240
_PER_TILE = _NBINS // 16


_EPW_PAD = 10240


@functools.cache
def _make_sc_histogram():
    mesh = plsc.VectorSubcoreMesh(core_axis_name="c", subcore_axis_name="s")
    return functools.partial(
        pl.kernel,
        out_type=(jax.ShapeDtypeStruct((1, _NBINS), jnp.float32),
                  jax.ShapeDtypeStruct((1, _NBINS), jnp.float32)),
        mesh=mesh,
        scratch_types=[
            pltpu.VMEM((_EPW_PAD,), jnp.int32),
            pltpu.VMEM((_EPW_PAD,), jnp.float32),
            pltpu.VMEM((_PER_TILE,), jnp.float32),
            pltpu.VMEM_SHARED((_NBINS,), jnp.float32),
        ],
    )(_sc_histogram_body)


def _sc_histogram_body(edge_hbm, out0_hbm, out1_hbm, idx_v, ones_v, zeros_v,
                       counts_sh):
    cid = lax.axis_index("c")
    sid = lax.axis_index("s")
    wid = sid * 2 + cid

    one16 = jnp.ones((16,), jnp.float32)
    zero16 = jnp.zeros((16,), jnp.float32)
    for i in range(_EPW_PAD // 16):
        ones_v[pl.ds(i * 16, 16)] = one16
    for i in range(_PER_TILE // 16):
        zeros_v[pl.ds(i * 16, 16)] = zero16
    lanes = lax.iota(jnp.int32, 16)
    for i in range((_EPW_PAD - _EPW) // 16):
        idx_v[pl.ds(_EPW + i * 16, 16)] = _N + 256 + i * 16 + lanes

    pltpu.sync_copy(zeros_v, counts_sh.at[pl.ds(sid * _PER_TILE, _PER_TILE)])
    pltpu.sync_copy(edge_hbm.at[pl.ds(wid * _EPW, _EPW)],
                    idx_v.at[pl.ds(0, _EPW)])
    plsc.subcore_barrier()

    pltpu.sync_copy(ones_v, counts_sh.at[idx_v], add=True)
    plsc.subcore_barrier()

    @pl.when(cid == 0)
    def _():
        pltpu.sync_copy(counts_sh.at[pl.ds(sid * _PER_TILE, _PER_TILE)],
                        out0_hbm.at[0, pl.ds(sid * _PER_TILE, _PER_TILE)])

    @pl.when(cid == 1)
    def _():
        pltpu.sync_copy(counts_sh.at[pl.ds(sid * _PER_TILE, _PER_TILE)],
                        out1_hbm.at[0, pl.ds(sid * _PER_TILE, _PER_TILE)])


def _tc_node_head(x_ref, w1_ref, woutt_ref, y_ref):
    hi = jax.lax.Precision.HIGHEST
    xv = x_ref[...].astype(jnp.bfloat16).astype(jnp.float32)
    w1 = w1_ref[...].astype(jnp.bfloat16).astype(jnp.float32)
    w_row = lax.dot_general(woutt_ref[...], w1, (((1,), (1,)), ((), ())),
                            precision=hi,
                            preferred_element_type=jnp.float32)
    y_ref[...] = lax.dot_general(w_row, xv, (((1,), (1,)), ((), ())),
                                 precision=hi,
                                 preferred_element_type=jnp.float32)


def _tc_final(c0_ref, c1_ref, y_ref, b1_ref, wout_ref, bout_ref, o_ref):
    hi = jax.lax.Precision.HIGHEST
    c1 = (c0_ref[...] + c1_ref[...])[:, :_N]
    out = lax.dot_general(c1, y_ref[...], (((1,), (1,)), ((), ())),
                          precision=hi, preferred_element_type=jnp.float32)
    bias = lax.dot_general(b1_ref[...], wout_ref[...],
                           (((1,), (1,)), ((), ())), precision=hi,
                           preferred_element_type=jnp.float32)
    o_ref[...] = out + jnp.float32(_N) * bias + bout_ref[...]


def kernel(x, edge_index, W1, b1, W_out, b_out):
    counts0, counts1 = _make_sc_histogram()(edge_index.reshape(2 * _E))

    y = pl.pallas_call(
        _tc_node_head,
        out_shape=jax.ShapeDtypeStruct((1, _N), jnp.float32),
    )(x, W1, W_out.reshape(1, _H))

    out = pl.pallas_call(
        _tc_final,
        out_shape=jax.ShapeDtypeStruct((1, 1), jnp.float32),
    )(counts0, counts1, y, b1.reshape(1, _H), W_out.reshape(1, _H),
      b_out.reshape(1, 1))
    return out
